# baseline (speedup 1.0000x reference)
import functools

import jax
import jax.numpy as jnp
from jax import lax
from jax.experimental import pallas as pl
from jax.experimental.pallas import tpu as pltpu
from jax.experimental.pallas import tpu_sc as plsc

_N = 10000
_D = 128
_E = 320000
_N_PAD = 10240
_ROWS_PER_TILE = _N_PAD // 16
_C = 128
_K = 32
_G = 5
_R = _G * _K
_E_PAD = 16 * _R * _C
_F32 = jnp.float32

_mesh = plsc.VectorSubcoreMesh(core_axis_name="c", subcore_axis_name="s")


@functools.partial(
    pl.kernel,
    out_type=jax.ShapeDtypeStruct((2, _N_PAD), _F32),
    mesh=_mesh,
    scratch_types=[
        pltpu.VMEM((_R, _C), jnp.int32),
        pltpu.VMEM((_C,), _F32),
        pltpu.VMEM_SHARED((_N_PAD,), _F32),
        pltpu.SemaphoreType.DMA,
        pltpu.SemaphoreType.DMA,
    ],
)
def _sc_degree(dst_hbm, deg_hbm, idx_v, ones_v, deg_sh, sem_a, sem_b):
    c = lax.axis_index("c")
    s = lax.axis_index("s")
    for k in range(_C // 16):
        ones_v[pl.ds(k * 16, 16)] = jnp.full((16,), 1.0, _F32)
    pltpu.sync_copy(dst_hbm.at[c, s], idx_v)
    for k in range(_ROWS_PER_TILE // _C):
        pltpu.sync_copy(ones_v, deg_sh.at[pl.ds(s * _ROWS_PER_TILE + k * _C, _C)])
    plsc.subcore_barrier()
    sems = (sem_a, sem_b)

    def chunk2(i, carry):
        d0 = pltpu.async_copy(ones_v, deg_sh.at[idx_v.at[2 * i]], sems[0],
                              add=True)
        d1 = pltpu.async_copy(ones_v, deg_sh.at[idx_v.at[2 * i + 1]], sems[1],
                              add=True)
        d0.wait()
        d1.wait()
        return carry

    lax.fori_loop(0, _R // 2, chunk2, 0)
    plsc.subcore_barrier()
    pltpu.sync_copy(deg_sh.at[pl.ds(s * _ROWS_PER_TILE, _ROWS_PER_TILE)],
                    deg_hbm.at[c, pl.ds(s * _ROWS_PER_TILE, _ROWS_PER_TILE)])


@functools.partial(
    pl.kernel,
    out_type=(jax.ShapeDtypeStruct((_N_PAD, _D), _F32),
              jax.ShapeDtypeStruct((_N_PAD, _D), _F32)),
    mesh=_mesh,
    scratch_types=[
        pltpu.VMEM((_K, _C), jnp.int32),
        pltpu.VMEM((_K, _C), jnp.int32),
        pltpu.VMEM((_C, _D), _F32),
        pltpu.VMEM((_C, _D), _F32),
        pltpu.VMEM_SHARED((_N_PAD, _D), _F32),
        pltpu.SemaphoreType.DMA,
        pltpu.SemaphoreType.DMA,
        pltpu.SemaphoreType.DMA,
        pltpu.SemaphoreType.DMA,
    ],
)
def _sc_aggregate(xs0_hbm, xs1_hbm, src_hbm, dst_hbm, agg0_hbm, agg1_hbm,
                  src_v, dst_v, rows_a, rows_b, acc_sh,
                  sem_a, sem_b, ssem_a, ssem_b):
    c = lax.axis_index("c")
    s = lax.axis_index("s")
    bufs = (rows_a, rows_b)
    sems = (sem_a, sem_b)
    ssems = (ssem_a, ssem_b)

    def run(xs_hbm, agg_hbm):
        row0 = s * _ROWS_PER_TILE
        pltpu.sync_copy(xs_hbm.at[pl.ds(row0, _ROWS_PER_TILE)],
                        acc_sh.at[pl.ds(row0, _ROWS_PER_TILE)])
        plsc.subcore_barrier()

        def group(g, carry):
            pltpu.sync_copy(src_hbm.at[c, s, pl.ds(g * _K, _K)], src_v)
            pltpu.sync_copy(dst_hbm.at[c, s, pl.ds(g * _K, _K)], dst_v)
            gd = [pltpu.async_copy(xs_hbm.at[src_v.at[0]], bufs[0], sems[0]),
                  None]
            sd = [None, None]
            for j in range(_K):
                b = j % 2
                gd[b].wait()
                sd[b] = pltpu.async_copy(bufs[b], acc_sh.at[dst_v.at[j]],
                                         ssems[b], add=True)
                if j >= 1:
                    sd[1 - b].wait()
                if j + 1 < _K:
                    gd[1 - b] = pltpu.async_copy(xs_hbm.at[src_v.at[j + 1]],
                                                 bufs[1 - b], sems[1 - b])
            sd[(_K - 1) % 2].wait()
            return carry

        lax.fori_loop(0, _G, group, 0)
        plsc.subcore_barrier()
        pltpu.sync_copy(acc_sh.at[pl.ds(row0, _ROWS_PER_TILE)],
                        agg_hbm.at[pl.ds(row0, _ROWS_PER_TILE)])

    @pl.when(c == 0)
    def _():
        run(xs0_hbm, agg0_hbm)

    @pl.when(c == 1)
    def _():
        run(xs1_hbm, agg1_hbm)


def _tc_prep(x_pad, deg0, deg1):
    def body(x_ref, d0_ref, d1_ref, xs0_ref, xs1_ref):
        xv = x_ref[...]
        xs0_ref[...] = xv * lax.rsqrt(d0_ref[...])
        xs1_ref[...] = xv * lax.rsqrt(d1_ref[...])

    return pl.pallas_call(
        body,
        out_shape=(jax.ShapeDtypeStruct((_N_PAD, _D), _F32),
                   jax.ShapeDtypeStruct((_N_PAD, _D), _F32)),
    )(x_pad, deg0, deg1)


def _tc_finish(agg0, agg1, deg0, deg1, W0, b0, W1, b1):
    def body(a0_ref, a1_ref, d0_ref, d1_ref, w0_ref, b0_ref, w1_ref, b1_ref,
             out_ref):
        h0 = jnp.dot(lax.rsqrt(d0_ref[...]) * a0_ref[...], w0_ref[...],
                     preferred_element_type=_F32,
                     precision=lax.Precision.HIGHEST) + b0_ref[...]
        h1 = jnp.dot(lax.rsqrt(d1_ref[...]) * a1_ref[...], w1_ref[...],
                     preferred_element_type=_F32,
                     precision=lax.Precision.HIGHEST) + b1_ref[...]
        out = jnp.maximum(h0, 0.0) + jnp.maximum(h1, 0.0)
        nrm = jnp.sqrt(jnp.sum(out * out, axis=1, keepdims=True))
        out_ref[...] = out / jnp.maximum(nrm, 1e-12)

    return pl.pallas_call(
        body,
        out_shape=jax.ShapeDtypeStruct((_N_PAD, _D), _F32),
    )(agg0, agg1, deg0, deg1, W0, b0, W1, b1)


def kernel(x, edge_index_0, edge_index_1, W0, b0, W1, b1):
    pad = _N + (jnp.arange(_E_PAD - _E, dtype=jnp.int32) % (_N_PAD - _N))

    def prep(ei):
        src = jnp.concatenate([ei[0], pad]).reshape(16, _R, _C)
        dst = jnp.concatenate([ei[1], pad]).reshape(16, _R, _C)
        return src, dst

    s0, d0 = prep(edge_index_0)
    s1, d1 = prep(edge_index_1)
    src = jnp.stack([s0, s1])
    dst = jnp.stack([d0, d1])

    deg = _sc_degree(dst)
    deg0 = deg[0].reshape(_N_PAD, 1)
    deg1 = deg[1].reshape(_N_PAD, 1)

    x_pad = jnp.pad(x, ((0, _N_PAD - _N), (0, 0)))
    xs0, xs1 = _tc_prep(x_pad, deg0, deg1)
    agg0, agg1 = _sc_aggregate(xs0, xs1, src, dst)
    out = _tc_finish(agg0, agg1, deg0, deg1, W0, b0, W1, b1)
    return out[:_N]


# speedup vs baseline: 1.1422x; 1.1422x over previous
import functools

import jax
import jax.numpy as jnp
from jax import lax
from jax.experimental import pallas as pl
from jax.experimental.pallas import tpu as pltpu
from jax.experimental.pallas import tpu_sc as plsc

_N = 10000
_D = 128
_E = 320000
_N_PAD = 10240
_ROWS_PER_TILE = _N_PAD // 16
_C = 128
_K = 32
_G = 5
_R = _G * _K
_E_PAD = 16 * _R * _C
_F32 = jnp.float32

_mesh = plsc.VectorSubcoreMesh(core_axis_name="c", subcore_axis_name="s")


@functools.partial(
    pl.kernel,
    out_type=jax.ShapeDtypeStruct((2, _N_PAD), _F32),
    mesh=_mesh,
    scratch_types=[
        pltpu.VMEM((_R, _C), jnp.int32),
        pltpu.VMEM((_C,), _F32),
        pltpu.VMEM_SHARED((_N_PAD,), _F32),
        pltpu.SemaphoreType.DMA,
        pltpu.SemaphoreType.DMA,
    ],
)
def _sc_degree(dst_hbm, deg_hbm, idx_v, ones_v, deg_sh, sem_a, sem_b):
    c = lax.axis_index("c")
    s = lax.axis_index("s")
    for k in range(_C // 16):
        ones_v[pl.ds(k * 16, 16)] = jnp.full((16,), 1.0, _F32)
    pltpu.sync_copy(dst_hbm.at[c, s], idx_v)
    for k in range(_ROWS_PER_TILE // _C):
        pltpu.sync_copy(ones_v, deg_sh.at[pl.ds(s * _ROWS_PER_TILE + k * _C, _C)])
    plsc.subcore_barrier()
    sems = (sem_a, sem_b)

    def chunk2(i, carry):
        d0 = pltpu.async_copy(ones_v, deg_sh.at[idx_v.at[2 * i]], sems[0],
                              add=True)
        d1 = pltpu.async_copy(ones_v, deg_sh.at[idx_v.at[2 * i + 1]], sems[1],
                              add=True)
        d0.wait()
        d1.wait()
        return carry

    lax.fori_loop(0, _R // 2, chunk2, 0)
    plsc.subcore_barrier()
    pltpu.sync_copy(deg_sh.at[pl.ds(s * _ROWS_PER_TILE, _ROWS_PER_TILE)],
                    deg_hbm.at[c, pl.ds(s * _ROWS_PER_TILE, _ROWS_PER_TILE)])


@functools.partial(
    pl.kernel,
    out_type=(jax.ShapeDtypeStruct((_N_PAD, _D), _F32),
              jax.ShapeDtypeStruct((_N_PAD, _D), _F32)),
    mesh=_mesh,
    scratch_types=[
        pltpu.VMEM((_K, _C), jnp.int32),
        pltpu.VMEM((_K, _C), jnp.int32),
        pltpu.VMEM((_C, _D), _F32),
        pltpu.VMEM((_C, _D), _F32),
        pltpu.VMEM_SHARED((_N_PAD, _D), _F32),
        pltpu.SemaphoreType.DMA,
        pltpu.SemaphoreType.DMA,
        pltpu.SemaphoreType.DMA,
        pltpu.SemaphoreType.DMA,
    ],
)
def _sc_aggregate(xs0_hbm, xs1_hbm, src_hbm, dst_hbm, agg0_hbm, agg1_hbm,
                  src_v, dst_v, rows_a, rows_b, acc_sh,
                  sem_a, sem_b, ssem_a, ssem_b):
    c = lax.axis_index("c")
    s = lax.axis_index("s")
    bufs = (rows_a, rows_b)
    sems = (sem_a, sem_b)
    ssems = (ssem_a, ssem_b)

    def run(xs_hbm, agg_hbm):
        row0 = s * _ROWS_PER_TILE
        pltpu.sync_copy(xs_hbm.at[pl.ds(row0, _ROWS_PER_TILE)],
                        acc_sh.at[pl.ds(row0, _ROWS_PER_TILE)])
        plsc.subcore_barrier()

        def group(g, carry):
            pltpu.sync_copy(src_hbm.at[c, s, pl.ds(g * _K, _K)], src_v)
            pltpu.sync_copy(dst_hbm.at[c, s, pl.ds(g * _K, _K)], dst_v)
            pending = pltpu.async_copy(xs_hbm.at[src_v.at[0]], bufs[0], sems[0])
            for j in range(_K):
                b = j % 2
                if j + 1 < _K:
                    nxt = pltpu.async_copy(xs_hbm.at[src_v.at[j + 1]],
                                           bufs[1 - b], sems[1 - b])
                pending.wait()
                pltpu.sync_copy(bufs[b], acc_sh.at[dst_v.at[j]], add=True)
                if j + 1 < _K:
                    pending = nxt
            return carry

        lax.fori_loop(0, _G, group, 0)
        plsc.subcore_barrier()
        pltpu.sync_copy(acc_sh.at[pl.ds(row0, _ROWS_PER_TILE)],
                        agg_hbm.at[pl.ds(row0, _ROWS_PER_TILE)])

    @pl.when(c == 0)
    def _():
        run(xs0_hbm, agg0_hbm)

    @pl.when(c == 1)
    def _():
        run(xs1_hbm, agg1_hbm)


def _tc_prep(x_pad, deg0, deg1):
    def body(x_ref, d0_ref, d1_ref, xs0_ref, xs1_ref):
        xv = x_ref[...]
        xs0_ref[...] = xv * lax.rsqrt(d0_ref[...])
        xs1_ref[...] = xv * lax.rsqrt(d1_ref[...])

    return pl.pallas_call(
        body,
        out_shape=(jax.ShapeDtypeStruct((_N_PAD, _D), _F32),
                   jax.ShapeDtypeStruct((_N_PAD, _D), _F32)),
    )(x_pad, deg0, deg1)


def _tc_finish(agg0, agg1, deg0, deg1, W0, b0, W1, b1):
    def body(a0_ref, a1_ref, d0_ref, d1_ref, w0_ref, b0_ref, w1_ref, b1_ref,
             out_ref):
        h0 = jnp.dot(lax.rsqrt(d0_ref[...]) * a0_ref[...], w0_ref[...],
                     preferred_element_type=_F32,
                     precision=lax.Precision.HIGHEST) + b0_ref[...]
        h1 = jnp.dot(lax.rsqrt(d1_ref[...]) * a1_ref[...], w1_ref[...],
                     preferred_element_type=_F32,
                     precision=lax.Precision.HIGHEST) + b1_ref[...]
        out = jnp.maximum(h0, 0.0) + jnp.maximum(h1, 0.0)
        nrm = jnp.sqrt(jnp.sum(out * out, axis=1, keepdims=True))
        out_ref[...] = out / jnp.maximum(nrm, 1e-12)

    return pl.pallas_call(
        body,
        out_shape=jax.ShapeDtypeStruct((_N_PAD, _D), _F32),
    )(agg0, agg1, deg0, deg1, W0, b0, W1, b1)


def kernel(x, edge_index_0, edge_index_1, W0, b0, W1, b1):
    pad = _N + (jnp.arange(_E_PAD - _E, dtype=jnp.int32) % (_N_PAD - _N))

    def prep(ei):
        src = jnp.concatenate([ei[0], pad]).reshape(16, _R, _C)
        dst = jnp.concatenate([ei[1], pad]).reshape(16, _R, _C)
        return src, dst

    s0, d0 = prep(edge_index_0)
    s1, d1 = prep(edge_index_1)
    src = jnp.stack([s0, s1])
    dst = jnp.stack([d0, d1])

    deg = _sc_degree(dst)
    deg0 = deg[0].reshape(_N_PAD, 1)
    deg1 = deg[1].reshape(_N_PAD, 1)

    x_pad = jnp.pad(x, ((0, _N_PAD - _N), (0, 0)))
    xs0, xs1 = _tc_prep(x_pad, deg0, deg1)
    agg0, agg1 = _sc_aggregate(xs0, xs1, src, dst)
    out = _tc_finish(agg0, agg1, deg0, deg1, W0, b0, W1, b1)
    return out[:_N]
